# Initial kernel scaffold; baseline (speedup 1.0000x reference)
import jax
import jax.numpy as jnp
from jax.experimental import pallas as pl


def kernel(x, edge_index):
    raise NotImplementedError("write your pallas kernel here")


# speedup vs baseline: 8.3398x; 8.3398x over previous
import functools

import jax
import jax.numpy as jnp
from jax import lax
from jax.experimental import pallas as pl
from jax.experimental.pallas import tpu as pltpu
from jax.experimental.pallas import tpu_sc as plsc

N = 10000
D = 128
DH = D // 2
E = 320000
EPS = 1e-8

NC, NS = 2, 16
NPAD = 10240
E_PER_T = E // NS
CHUNK = 128
N_FULL = E_PER_T // CHUNK
TAIL = E_PER_T - N_FULL * CHUNK

_mesh = plsc.VectorSubcoreMesh(
    core_axis_name="c", subcore_axis_name="s", num_cores=NC, num_subcores=NS)
_untiled = pltpu.CompilerParams(use_tc_tiling_on_sc=False)


def _deg_body(src_hbm, dst_hbm, z_hbm, outd_hbm, outs_hbm, idx, idx_t,
              ones_b, ones_t, deg_sh, sem):
    cid = lax.axis_index("c")
    sid = lax.axis_index("s")

    one = jnp.ones((16,), jnp.float32)

    def fill(r, _):
        ones_b[r, :] = one
        return _
    lax.fori_loop(0, CHUNK, fill, None)

    def fill_t(r, _):
        ones_t[r, :] = one
        return _
    lax.fori_loop(0, TAIL, fill_t, None)

    @pl.when(sid == 0)
    def _():
        pltpu.sync_copy(z_hbm, deg_sh)
    plsc.subcore_barrier()

    base0 = sid * E_PER_T

    def run(e_hbm):
        def chunk(c, _):
            base = base0 + c * CHUNK
            pltpu.sync_copy(e_hbm.at[pl.ds(base, CHUNK)], idx)
            pltpu.sync_copy(ones_b, deg_sh.at[idx], add=True)
            return _
        lax.fori_loop(0, N_FULL, chunk, None)
        tb = base0 + N_FULL * CHUNK
        pltpu.sync_copy(e_hbm.at[pl.ds(tb, TAIL)], idx_t)
        pltpu.sync_copy(ones_t, deg_sh.at[idx_t], add=True)

    @pl.when(cid == 0)
    def _():
        run(dst_hbm)

    @pl.when(cid == 1)
    def _():
        run(src_hbm)

    plsc.subcore_barrier()

    @pl.when(jnp.logical_and(cid == 0, sid == 0))
    def _():
        pltpu.sync_copy(deg_sh, outd_hbm)

    @pl.when(jnp.logical_and(cid == 1, sid == 0))
    def _():
        pltpu.sync_copy(deg_sh, outs_hbm)


_deg_kernel = functools.partial(
    pl.kernel,
    out_type=(jax.ShapeDtypeStruct((NPAD, 16), jnp.float32),
              jax.ShapeDtypeStruct((NPAD, 16), jnp.float32)),
    mesh=_mesh,
    scratch_types=[
        pltpu.VMEM((CHUNK,), jnp.int32),
        pltpu.VMEM((TAIL,), jnp.int32),
        pltpu.VMEM((CHUNK, 16), jnp.float32),
        pltpu.VMEM((TAIL, 16), jnp.float32),
        pltpu.VMEM_SHARED((NPAD, 16), jnp.float32),
        pltpu.SemaphoreType.DMA,
    ],
    compiler_params=_untiled,
)(_deg_body)


def _spmm_body(g0_hbm, g1_hbm, src_hbm, dst_hbm, z_hbm, out0_hbm, out1_hbm,
               idxs, idxd, idxs_t, idxd_t, rows, rows_t, acc_sh, sem):
    cid = lax.axis_index("c")
    sid = lax.axis_index("s")

    @pl.when(sid == 0)
    def _():
        pltpu.sync_copy(z_hbm, acc_sh)
    plsc.subcore_barrier()

    base0 = sid * E_PER_T

    def run(g_hbm):
        def chunk(c, _):
            base = base0 + c * CHUNK
            pltpu.sync_copy(src_hbm.at[pl.ds(base, CHUNK)], idxs)
            pltpu.sync_copy(dst_hbm.at[pl.ds(base, CHUNK)], idxd)
            pltpu.async_copy(g_hbm.at[idxs], rows, sem).wait()
            pltpu.sync_copy(rows, acc_sh.at[idxd], add=True)
            return _
        lax.fori_loop(0, N_FULL, chunk, None)
        tb = base0 + N_FULL * CHUNK
        pltpu.sync_copy(src_hbm.at[pl.ds(tb, TAIL)], idxs_t)
        pltpu.sync_copy(dst_hbm.at[pl.ds(tb, TAIL)], idxd_t)
        pltpu.async_copy(g_hbm.at[idxs_t], rows_t, sem).wait()
        pltpu.sync_copy(rows_t, acc_sh.at[idxd_t], add=True)

    @pl.when(cid == 0)
    def _():
        run(g0_hbm)

    @pl.when(cid == 1)
    def _():
        run(g1_hbm)

    plsc.subcore_barrier()

    @pl.when(jnp.logical_and(cid == 0, sid == 0))
    def _():
        pltpu.sync_copy(acc_sh, out0_hbm)

    @pl.when(jnp.logical_and(cid == 1, sid == 0))
    def _():
        pltpu.sync_copy(acc_sh, out1_hbm)


_spmm_kernel = functools.partial(
    pl.kernel,
    out_type=(jax.ShapeDtypeStruct((NPAD, DH), jnp.float32),
              jax.ShapeDtypeStruct((NPAD, DH), jnp.float32)),
    mesh=_mesh,
    scratch_types=[
        pltpu.VMEM((CHUNK,), jnp.int32),
        pltpu.VMEM((CHUNK,), jnp.int32),
        pltpu.VMEM((TAIL,), jnp.int32),
        pltpu.VMEM((TAIL,), jnp.int32),
        pltpu.VMEM((CHUNK, DH), jnp.float32),
        pltpu.VMEM((TAIL, DH), jnp.float32),
        pltpu.VMEM_SHARED((NPAD, DH), jnp.float32),
        pltpu.SemaphoreType.DMA,
    ],
    compiler_params=_untiled,
)(_spmm_body)


def _prep_body(degd_ref, degs_ref, x_ref, g0_ref, g1_ref, ab_ref):
    dr = degd_ref[:N, 0]
    dc = degs_ref[:N, 0]
    a = 1.0 / (jnp.sqrt(dr) + EPS)
    b = 1.0 / (jnp.sqrt(dc) + EPS)
    ab_ref[0, :] = a
    ab_ref[1, :] = b
    g = x_ref[...] * b[:, None]
    g0_ref[...] = g[:, :DH]
    g1_ref[...] = g[:, DH:]


def _prep(degd, degs, x):
    return pl.pallas_call(
        _prep_body,
        out_shape=(jax.ShapeDtypeStruct((N, DH), jnp.float32),
                   jax.ShapeDtypeStruct((N, DH), jnp.float32),
                   jax.ShapeDtypeStruct((2, N), jnp.float32)),
    )(degd, degs, x)


def _mid_body(acc0_ref, acc1_ref, ab_ref, x_ref, g0_ref, g1_ref, op_ref):
    s = jnp.concatenate([acc0_ref[:N, :], acc1_ref[:N, :]], axis=1)
    a = ab_ref[0, :]
    b = ab_ref[1, :]
    f1 = s * (a * 0.5)[:, None]
    nrm = jnp.sqrt(jnp.sum(f1 * f1, axis=1, keepdims=True))
    op_ref[...] = x_ref[...] + f1 / jnp.maximum(nrm, 1e-12)
    g2 = f1 * b[:, None]
    g0_ref[...] = g2[:, :DH]
    g1_ref[...] = g2[:, DH:]


def _mid(acc0, acc1, ab, x):
    return pl.pallas_call(
        _mid_body,
        out_shape=(jax.ShapeDtypeStruct((N, DH), jnp.float32),
                   jax.ShapeDtypeStruct((N, DH), jnp.float32),
                   jax.ShapeDtypeStruct((N, D), jnp.float32)),
    )(acc0, acc1, ab, x)


def _fin_body(acc0_ref, acc1_ref, ab_ref, op_ref, out_ref):
    s = jnp.concatenate([acc0_ref[:N, :], acc1_ref[:N, :]], axis=1)
    a = ab_ref[0, :]
    f2 = s * a[:, None]
    nrm = jnp.sqrt(jnp.sum(f2 * f2, axis=1, keepdims=True))
    out_ref[...] = op_ref[...] + f2 / jnp.maximum(nrm, 1e-12)


def _fin(acc0, acc1, ab, op):
    return pl.pallas_call(
        _fin_body,
        out_shape=jax.ShapeDtypeStruct((N, D), jnp.float32),
    )(acc0, acc1, ab, op)


def kernel(x, edge_index):
    src = edge_index[0]
    dst = edge_index[1]
    z16 = jnp.zeros((NPAD, 16), jnp.float32)
    zDH = jnp.zeros((NPAD, DH), jnp.float32)
    degd, degs = _deg_kernel(src, dst, z16)
    g0, g1, ab = _prep(degd, degs, x)
    a10, a11 = _spmm_kernel(g0, g1, src, dst, zDH)
    h0, h1, out_part = _mid(a10, a11, ab, x)
    a20, a21 = _spmm_kernel(h0, h1, src, dst, zDH)
    return _fin(a20, a21, ab, out_part)


# speedup vs baseline: 17.2375x; 2.0669x over previous
import functools

import jax
import jax.numpy as jnp
from jax import lax
from jax.experimental import pallas as pl
from jax.experimental.pallas import tpu as pltpu
from jax.experimental.pallas import tpu_sc as plsc

N = 10000
D = 128
DH = D // 2
E = 320000
EPS = 1e-8

NC, NS = 2, 16
NPAD = 10240
CHUNK = 128
EROWS = E // CHUNK
RPT_ROWS = EROWS // NS
EXTRA = EROWS - RPT_ROWS * NS
G = 3
PB = 2
NIT = RPT_ROWS // (G * PB)

_mesh = plsc.VectorSubcoreMesh(
    core_axis_name="c", subcore_axis_name="s", num_cores=NC, num_subcores=NS)
_untiled = pltpu.CompilerParams(use_tc_tiling_on_sc=False)


def _deg_body(src2_hbm, dst2_hbm, z_hbm, outd_hbm, outs_hbm,
              idx, ones_b, deg_sh, sem):
    cid = lax.axis_index("c")
    sid = lax.axis_index("s")

    one = jnp.ones((16,), jnp.float32)

    def fill(r, _):
        ones_b[r, :] = one
        return _
    lax.fori_loop(0, CHUNK, fill, None)

    @pl.when(sid == 0)
    def _():
        pltpu.sync_copy(z_hbm, deg_sh)
    plsc.subcore_barrier()

    row0 = sid * RPT_ROWS + jnp.minimum(sid, EXTRA)

    def run(e2_hbm):
        def it(k, _):
            gb = row0 + k * (G * PB)
            sds = []
            for p in range(PB):
                pltpu.sync_copy(e2_hbm.at[pl.ds(gb + p * G, G)],
                                idx.at[pl.ds(p * G, G)])
                for j in range(G):
                    sds.append(pltpu.async_copy(
                        ones_b, deg_sh.at[idx.at[p * G + j]], sem, add=True))
            for d in sds:
                d.wait()
            return _
        lax.fori_loop(0, NIT, it, None)

        @pl.when(sid < EXTRA)
        def _():
            er = row0 + RPT_ROWS
            pltpu.sync_copy(e2_hbm.at[pl.ds(er, 1)], idx.at[pl.ds(0, 1)])
            pltpu.sync_copy(ones_b, deg_sh.at[idx.at[0]], add=True)

    @pl.when(cid == 0)
    def _():
        run(dst2_hbm)

    @pl.when(cid == 1)
    def _():
        run(src2_hbm)

    plsc.subcore_barrier()

    @pl.when(jnp.logical_and(cid == 0, sid == 0))
    def _():
        pltpu.sync_copy(deg_sh, outd_hbm)

    @pl.when(jnp.logical_and(cid == 1, sid == 0))
    def _():
        pltpu.sync_copy(deg_sh, outs_hbm)


_deg_kernel = functools.partial(
    pl.kernel,
    out_type=(jax.ShapeDtypeStruct((NPAD, 16), jnp.float32),
              jax.ShapeDtypeStruct((NPAD, 16), jnp.float32)),
    mesh=_mesh,
    scratch_types=[
        pltpu.VMEM((PB * G, CHUNK), jnp.int32),
        pltpu.VMEM((CHUNK, 16), jnp.float32),
        pltpu.VMEM_SHARED((NPAD, 16), jnp.float32),
        pltpu.SemaphoreType.DMA,
    ],
    compiler_params=_untiled,
)(_deg_body)


def _spmm_body(g0_hbm, g1_hbm, src2_hbm, dst2_hbm, z_hbm, out0_hbm, out1_hbm,
               idxs, idxd, rows, acc_sh, sg0, sg1, sg2, sem_s):
    cid = lax.axis_index("c")
    sid = lax.axis_index("s")
    sg = [sg0, sg1, sg2]

    @pl.when(sid == 0)
    def _():
        pltpu.sync_copy(z_hbm, acc_sh)
    plsc.subcore_barrier()

    row0 = sid * RPT_ROWS + jnp.minimum(sid, EXTRA)

    def run(g_hbm):
        def it(k, _):
            gb = row0 + k * (G * PB)
            gds = []
            for p in range(PB):
                pltpu.sync_copy(src2_hbm.at[pl.ds(gb + p * G, G)],
                                idxs.at[pl.ds(p * G, G)])
                pltpu.sync_copy(dst2_hbm.at[pl.ds(gb + p * G, G)],
                                idxd.at[pl.ds(p * G, G)])
                for j in range(G):
                    c = p * G + j
                    gds.append(pltpu.async_copy(
                        g_hbm.at[idxs.at[c]],
                        rows.at[pl.ds(c * CHUNK, CHUNK)], sg[p]))
            sds = []
            for p in range(PB):
                for j in range(G):
                    gds[p * G + j].wait()
                for j in range(G):
                    c = p * G + j
                    sds.append(pltpu.async_copy(
                        rows.at[pl.ds(c * CHUNK, CHUNK)],
                        acc_sh.at[idxd.at[c]], sem_s, add=True))
            for d in sds:
                d.wait()
            return _
        lax.fori_loop(0, NIT, it, None)

        @pl.when(sid < EXTRA)
        def _():
            er = row0 + RPT_ROWS
            pltpu.sync_copy(src2_hbm.at[pl.ds(er, 1)], idxs.at[pl.ds(0, 1)])
            pltpu.sync_copy(dst2_hbm.at[pl.ds(er, 1)], idxd.at[pl.ds(0, 1)])
            pltpu.async_copy(g_hbm.at[idxs.at[0]],
                             rows.at[pl.ds(0, CHUNK)], sg0).wait()
            pltpu.sync_copy(rows.at[pl.ds(0, CHUNK)],
                            acc_sh.at[idxd.at[0]], add=True)

    @pl.when(cid == 0)
    def _():
        run(g0_hbm)

    @pl.when(cid == 1)
    def _():
        run(g1_hbm)

    plsc.subcore_barrier()

    @pl.when(jnp.logical_and(cid == 0, sid == 0))
    def _():
        pltpu.sync_copy(acc_sh, out0_hbm)

    @pl.when(jnp.logical_and(cid == 1, sid == 0))
    def _():
        pltpu.sync_copy(acc_sh, out1_hbm)


_spmm_kernel = functools.partial(
    pl.kernel,
    out_type=(jax.ShapeDtypeStruct((NPAD, DH), jnp.float32),
              jax.ShapeDtypeStruct((NPAD, DH), jnp.float32)),
    mesh=_mesh,
    scratch_types=[
        pltpu.VMEM((PB * G, CHUNK), jnp.int32),
        pltpu.VMEM((PB * G, CHUNK), jnp.int32),
        pltpu.VMEM((PB * G * CHUNK, DH), jnp.float32),
        pltpu.VMEM_SHARED((NPAD, DH), jnp.float32),
        pltpu.SemaphoreType.DMA,
        pltpu.SemaphoreType.DMA,
        pltpu.SemaphoreType.DMA,
        pltpu.SemaphoreType.DMA,
    ],
    compiler_params=_untiled,
)(_spmm_body)


def _prep_body(degd_ref, degs_ref, x_ref, g0_ref, g1_ref, ab_ref):
    dr = degd_ref[:N, 0]
    dc = degs_ref[:N, 0]
    a = 1.0 / (jnp.sqrt(dr) + EPS)
    b = 1.0 / (jnp.sqrt(dc) + EPS)
    ab_ref[0, :] = a
    ab_ref[1, :] = b
    g = x_ref[...] * b[:, None]
    g0_ref[...] = g[:, :DH]
    g1_ref[...] = g[:, DH:]


def _prep(degd, degs, x):
    return pl.pallas_call(
        _prep_body,
        out_shape=(jax.ShapeDtypeStruct((N, DH), jnp.float32),
                   jax.ShapeDtypeStruct((N, DH), jnp.float32),
                   jax.ShapeDtypeStruct((2, N), jnp.float32)),
    )(degd, degs, x)


def _mid_body(acc0_ref, acc1_ref, ab_ref, x_ref, g0_ref, g1_ref, op_ref):
    s = jnp.concatenate([acc0_ref[:N, :], acc1_ref[:N, :]], axis=1)
    a = ab_ref[0, :]
    b = ab_ref[1, :]
    f1 = s * (a * 0.5)[:, None]
    nrm = jnp.sqrt(jnp.sum(f1 * f1, axis=1, keepdims=True))
    op_ref[...] = x_ref[...] + f1 / jnp.maximum(nrm, 1e-12)
    g2 = f1 * b[:, None]
    g0_ref[...] = g2[:, :DH]
    g1_ref[...] = g2[:, DH:]


def _mid(acc0, acc1, ab, x):
    return pl.pallas_call(
        _mid_body,
        out_shape=(jax.ShapeDtypeStruct((N, DH), jnp.float32),
                   jax.ShapeDtypeStruct((N, DH), jnp.float32),
                   jax.ShapeDtypeStruct((N, D), jnp.float32)),
    )(acc0, acc1, ab, x)


def _fin_body(acc0_ref, acc1_ref, ab_ref, op_ref, out_ref):
    s = jnp.concatenate([acc0_ref[:N, :], acc1_ref[:N, :]], axis=1)
    a = ab_ref[0, :]
    f2 = s * a[:, None]
    nrm = jnp.sqrt(jnp.sum(f2 * f2, axis=1, keepdims=True))
    out_ref[...] = op_ref[...] + f2 / jnp.maximum(nrm, 1e-12)


def _fin(acc0, acc1, ab, op):
    return pl.pallas_call(
        _fin_body,
        out_shape=jax.ShapeDtypeStruct((N, D), jnp.float32),
    )(acc0, acc1, ab, op)


def kernel(x, edge_index):
    src2 = edge_index[0].reshape(EROWS, CHUNK)
    dst2 = edge_index[1].reshape(EROWS, CHUNK)
    z16 = jnp.zeros((NPAD, 16), jnp.float32)
    zDH = jnp.zeros((NPAD, DH), jnp.float32)
    degd, degs = _deg_kernel(src2, dst2, z16)
    g0, g1, ab = _prep(degd, degs, x)
    a10, a11 = _spmm_kernel(g0, g1, src2, dst2, zDH)
    h0, h1, out_part = _mid(a10, a11, ab, x)
    a20, a21 = _spmm_kernel(h0, h1, src2, dst2, zDH)
    return _fin(a20, a21, ab, out_part)


# speedup vs baseline: 17.6458x; 1.0237x over previous
import functools

import jax
import jax.numpy as jnp
from jax import lax
from jax.experimental import pallas as pl
from jax.experimental.pallas import tpu as pltpu
from jax.experimental.pallas import tpu_sc as plsc

N = 10000
D = 128
DH = D // 2
E = 320000
EPS = 1e-8

NC, NS = 2, 16
NPAD = 10240
CHUNK = 128
EROWS = E // CHUNK
RPT_ROWS = EROWS // NS
EXTRA = EROWS - RPT_ROWS * NS
G = 3
PB = 2
NIT = RPT_ROWS // (G * PB)

_mesh = plsc.VectorSubcoreMesh(
    core_axis_name="c", subcore_axis_name="s", num_cores=NC, num_subcores=NS)
_untiled = pltpu.CompilerParams(use_tc_tiling_on_sc=False)


def _deg_body(src2_hbm, dst2_hbm, z_hbm, outd_hbm, outs_hbm,
              idx, ones_b, deg_sh, sem):
    cid = lax.axis_index("c")
    sid = lax.axis_index("s")

    one = jnp.ones((16,), jnp.float32)

    def fill(r, _):
        ones_b[r, :] = one
        return _
    lax.fori_loop(0, CHUNK, fill, None)

    @pl.when(sid == 0)
    def _():
        pltpu.sync_copy(z_hbm, deg_sh)
    plsc.subcore_barrier()

    row0 = sid * RPT_ROWS + jnp.minimum(sid, EXTRA)

    def run(e2_hbm):
        def it(k, _):
            gb = row0 + k * (G * PB)
            sds = []
            for p in range(PB):
                pltpu.sync_copy(e2_hbm.at[pl.ds(gb + p * G, G)],
                                idx.at[pl.ds(p * G, G)])
                for j in range(G):
                    sds.append(pltpu.async_copy(
                        ones_b, deg_sh.at[idx.at[p * G + j]], sem, add=True))
            for d in sds:
                d.wait()
            return _
        lax.fori_loop(0, NIT, it, None)

        @pl.when(sid < EXTRA)
        def _():
            er = row0 + RPT_ROWS
            pltpu.sync_copy(e2_hbm.at[pl.ds(er, 1)], idx.at[pl.ds(0, 1)])
            pltpu.sync_copy(ones_b, deg_sh.at[idx.at[0]], add=True)

    @pl.when(cid == 0)
    def _():
        run(dst2_hbm)

    @pl.when(cid == 1)
    def _():
        run(src2_hbm)

    plsc.subcore_barrier()

    @pl.when(jnp.logical_and(cid == 0, sid == 0))
    def _():
        pltpu.sync_copy(deg_sh, outd_hbm)

    @pl.when(jnp.logical_and(cid == 1, sid == 0))
    def _():
        pltpu.sync_copy(deg_sh, outs_hbm)


_deg_kernel = functools.partial(
    pl.kernel,
    out_type=(jax.ShapeDtypeStruct((NPAD, 16), jnp.float32),
              jax.ShapeDtypeStruct((NPAD, 16), jnp.float32)),
    mesh=_mesh,
    scratch_types=[
        pltpu.VMEM((PB * G, CHUNK), jnp.int32),
        pltpu.VMEM((CHUNK, 16), jnp.float32),
        pltpu.VMEM_SHARED((NPAD, 16), jnp.float32),
        pltpu.SemaphoreType.DMA,
    ],
    compiler_params=_untiled,
)(_deg_body)


def _spmm_body(g0_hbm, g1_hbm, src2_hbm, dst2_hbm, z_hbm, out0_hbm, out1_hbm,
               idxs, idxd, rows, acc_sh, sg0, sg1, sg2, sem_s):
    cid = lax.axis_index("c")
    sid = lax.axis_index("s")
    sg = [sg0, sg1, sg2]

    @pl.when(sid == 0)
    def _():
        pltpu.sync_copy(z_hbm, acc_sh)
    plsc.subcore_barrier()

    row0 = sid * RPT_ROWS + jnp.minimum(sid, EXTRA)

    def run(g_hbm):
        def it(k, _):
            gb = row0 + k * (G * PB)
            gds = []
            for p in range(PB):
                pltpu.sync_copy(src2_hbm.at[pl.ds(gb + p * G, G)],
                                idxs.at[pl.ds(p * G, G)])
                pltpu.sync_copy(dst2_hbm.at[pl.ds(gb + p * G, G)],
                                idxd.at[pl.ds(p * G, G)])
                for j in range(G):
                    c = p * G + j
                    gds.append(pltpu.async_copy(
                        g_hbm.at[idxs.at[c]],
                        rows.at[pl.ds(c * CHUNK, CHUNK)], sg[p]))
            sds = []
            for p in range(PB):
                for j in range(G):
                    gds[p * G + j].wait()
                for j in range(G):
                    c = p * G + j
                    sds.append(pltpu.async_copy(
                        rows.at[pl.ds(c * CHUNK, CHUNK)],
                        acc_sh.at[idxd.at[c]], sem_s, add=True))
            for d in sds:
                d.wait()
            return _
        lax.fori_loop(0, NIT, it, None)

        @pl.when(sid < EXTRA)
        def _():
            er = row0 + RPT_ROWS
            pltpu.sync_copy(src2_hbm.at[pl.ds(er, 1)], idxs.at[pl.ds(0, 1)])
            pltpu.sync_copy(dst2_hbm.at[pl.ds(er, 1)], idxd.at[pl.ds(0, 1)])
            pltpu.async_copy(g_hbm.at[idxs.at[0]],
                             rows.at[pl.ds(0, CHUNK)], sg0).wait()
            pltpu.sync_copy(rows.at[pl.ds(0, CHUNK)],
                            acc_sh.at[idxd.at[0]], add=True)

    @pl.when(cid == 0)
    def _():
        run(g0_hbm)

    @pl.when(cid == 1)
    def _():
        run(g1_hbm)

    plsc.subcore_barrier()

    @pl.when(jnp.logical_and(cid == 0, sid == 0))
    def _():
        pltpu.sync_copy(acc_sh, out0_hbm)

    @pl.when(jnp.logical_and(cid == 1, sid == 0))
    def _():
        pltpu.sync_copy(acc_sh, out1_hbm)


_spmm_kernel = functools.partial(
    pl.kernel,
    out_type=(jax.ShapeDtypeStruct((NPAD, DH), jnp.float32),
              jax.ShapeDtypeStruct((NPAD, DH), jnp.float32)),
    mesh=_mesh,
    scratch_types=[
        pltpu.VMEM((PB * G, CHUNK), jnp.int32),
        pltpu.VMEM((PB * G, CHUNK), jnp.int32),
        pltpu.VMEM((PB * G * CHUNK, DH), jnp.float32),
        pltpu.VMEM_SHARED((NPAD, DH), jnp.float32),
        pltpu.SemaphoreType.DMA,
        pltpu.SemaphoreType.DMA,
        pltpu.SemaphoreType.DMA,
        pltpu.SemaphoreType.DMA,
    ],
    compiler_params=_untiled,
)(_spmm_body)


NB = 1000
_GRID = N // NB

_bN = pl.BlockSpec((NB, DH), lambda i: (i, 0))
_bX = pl.BlockSpec((NB, D), lambda i: (i, 0))
_bAB = pl.BlockSpec((NB, 2), lambda i: (i, 0))
_b16 = pl.BlockSpec((NB, 16), lambda i: (i, 0))


def _prep_body(degd_ref, degs_ref, x_ref, g0_ref, g1_ref, ab_ref):
    dr = degd_ref[:, 0]
    dc = degs_ref[:, 0]
    a = 1.0 / (jnp.sqrt(dr) + EPS)
    b = 1.0 / (jnp.sqrt(dc) + EPS)
    ab_ref[:, 0] = a
    ab_ref[:, 1] = b
    x = x_ref[...]
    g0_ref[...] = x[:, :DH] * b[:, None]
    g1_ref[...] = x[:, DH:] * b[:, None]


def _prep(degd, degs, x):
    return pl.pallas_call(
        _prep_body,
        grid=(_GRID,),
        in_specs=[_b16, _b16, _bX],
        out_specs=(_bN, _bN, _bAB),
        out_shape=(jax.ShapeDtypeStruct((N, DH), jnp.float32),
                   jax.ShapeDtypeStruct((N, DH), jnp.float32),
                   jax.ShapeDtypeStruct((N, 2), jnp.float32)),
    )(degd, degs, x)


def _mid_body(acc0_ref, acc1_ref, ab_ref, x_ref, g0_ref, g1_ref, op_ref):
    s0 = acc0_ref[...]
    s1 = acc1_ref[...]
    a = ab_ref[:, 0]
    b = ab_ref[:, 1]
    f0 = s0 * (a * 0.5)[:, None]
    f1 = s1 * (a * 0.5)[:, None]
    n2 = jnp.sum(f0 * f0, axis=1, keepdims=True) + \
        jnp.sum(f1 * f1, axis=1, keepdims=True)
    inv = 1.0 / jnp.maximum(jnp.sqrt(n2), 1e-12)
    op_ref[...] = x_ref[...] + jnp.concatenate([f0 * inv, f1 * inv], axis=1)
    g0_ref[...] = f0 * b[:, None]
    g1_ref[...] = f1 * b[:, None]


def _mid(acc0, acc1, ab, x):
    return pl.pallas_call(
        _mid_body,
        grid=(_GRID,),
        in_specs=[_bN, _bN, _bAB, _bX],
        out_specs=(_bN, _bN, _bX),
        out_shape=(jax.ShapeDtypeStruct((N, DH), jnp.float32),
                   jax.ShapeDtypeStruct((N, DH), jnp.float32),
                   jax.ShapeDtypeStruct((N, D), jnp.float32)),
    )(acc0, acc1, ab, x)


def _fin_body(acc0_ref, acc1_ref, ab_ref, op_ref, out_ref):
    a = ab_ref[:, 0]
    f0 = acc0_ref[...] * a[:, None]
    f1 = acc1_ref[...] * a[:, None]
    n2 = jnp.sum(f0 * f0, axis=1, keepdims=True) + \
        jnp.sum(f1 * f1, axis=1, keepdims=True)
    inv = 1.0 / jnp.maximum(jnp.sqrt(n2), 1e-12)
    out_ref[...] = op_ref[...] + jnp.concatenate([f0 * inv, f1 * inv], axis=1)


def _fin(acc0, acc1, ab, op):
    return pl.pallas_call(
        _fin_body,
        grid=(_GRID,),
        in_specs=[_bN, _bN, _bAB, _bX],
        out_specs=_bX,
        out_shape=jax.ShapeDtypeStruct((N, D), jnp.float32),
    )(acc0, acc1, ab, op)


def kernel(x, edge_index):
    src2 = edge_index[0].reshape(EROWS, CHUNK)
    dst2 = edge_index[1].reshape(EROWS, CHUNK)
    z16 = jnp.zeros((NPAD, 16), jnp.float32)
    zDH = jnp.zeros((NPAD, DH), jnp.float32)
    degd, degs = _deg_kernel(src2, dst2, z16)
    g0, g1, ab = _prep(degd, degs, x)
    a10, a11 = _spmm_kernel(g0, g1, src2, dst2, zDH)
    h0, h1, out_part = _mid(a10, a11, ab, x)
    a20, a21 = _spmm_kernel(h0, h1, src2, dst2, zDH)
    return _fin(a20, a21, ab, out_part)
